# baseline (speedup 1.0000x reference)
import jax
import jax.numpy as jnp
from jax.experimental import pallas as pl
from jax.experimental.pallas import tpu as pltpu

B = 16384
K = 64
BLK = 1024


def _tc_body(gu, gi, gut, git, bu, bi, but, bit, mu, out):
    prod = gu[...] * gi[...] + gut[...] * git[...]
    s = jnp.sum(prod, axis=1)
    out[...] = (s + bu[...][:, 0] + bi[...][:, 0] + but[...][:, 0]
                + bit[...][:, 0] + mu[0, 0])


def kernel(gu, gi, gut, git, bu, bi, but, bit, Mu):
    grid = (B // BLK,)
    mat_spec = pl.BlockSpec((BLK, K), lambda i: (i, 0))
    bias_spec = pl.BlockSpec((BLK, 1), lambda i: (i, 0))
    mu_spec = pl.BlockSpec((1, 1), lambda i: (0, 0))
    out = pl.pallas_call(
        _tc_body,
        grid=grid,
        in_specs=[mat_spec, mat_spec, mat_spec, mat_spec,
                  bias_spec, bias_spec, bias_spec, bias_spec, mu_spec],
        out_specs=pl.BlockSpec((BLK,), lambda i: (i,)),
        out_shape=jax.ShapeDtypeStruct((B,), jnp.float32),
    )(gu, gi, gut, git, bu, bi, but, bit, Mu)
    return out
